# Initial kernel scaffold; baseline (speedup 1.0000x reference)
import jax
import jax.numpy as jnp
from jax.experimental import pallas as pl


def kernel(x, edge_index, edge_attr, Wself, Wnbr, Att1, Att2, bias, gamma, beta, Wc, bc):
    raise NotImplementedError("write your pallas kernel here")


# speedup vs baseline: 2.7794x; 2.7794x over previous
import functools

import jax
import jax.numpy as jnp
from jax import lax
from jax.experimental import pallas as pl
from jax.experimental.pallas import tpu as pltpu
from jax.experimental.pallas import tpu_sc as plsc

N = 10000
E = 320000
D = 128
EDIM = 16
L = 3
NC = 2
NS = 16
NW = NC * NS
EP = 327680
EPT = EP // NW
CH = 128
NCHK = EPT // CH
NPAD = 10240
BNODE = 400
GN = N // BNODE
BEDGE = 512
GE = EP // BEDGE

@functools.cache
def _mesh():
    return plsc.VectorSubcoreMesh(core_axis_name="c", subcore_axis_name="s")


def _tc_proj(h, wcat):
    def body(h_ref, w_ref, pd_ref, ps_ref, pn_ref, sx_ref):
        prod = jnp.dot(h_ref[...], w_ref[...], preferred_element_type=jnp.float32)
        pd_ref[...] = prod[:, 0:D]
        ps_ref[...] = prod[:, D:2 * D]
        pn_ref[...] = prod[:, 2 * D:3 * D]
        sx_ref[...] = prod[:, 3 * D:4 * D]

    return pl.pallas_call(
        body,
        grid=(GN,),
        in_specs=[pl.BlockSpec((BNODE, D), lambda i: (i, 0)),
                  pl.BlockSpec((D, 4 * D), lambda i: (0, 0))],
        out_specs=[pl.BlockSpec((BNODE, D), lambda i: (i, 0))] * 4,
        out_shape=[jax.ShapeDtypeStruct((N, D), jnp.float32)] * 4,
    )(h, wcat)


def _sc_gather(pd, ps, dstp, srcp):
    @functools.partial(
        pl.kernel,
        mesh=_mesh(),
        compiler_params=pltpu.CompilerParams(needs_layout_passes=False),
        out_type=[jax.ShapeDtypeStruct((EP, D), jnp.float32)] * 2,
        scratch_types=[
            pltpu.VMEM((CH,), jnp.int32),
            pltpu.VMEM((CH,), jnp.int32),
            pltpu.VMEM((CH, D), jnp.float32),
            pltpu.VMEM((CH, D), jnp.float32),
            pltpu.SemaphoreType.DMA,
            pltpu.SemaphoreType.DMA,
        ],
    )
    def k(pd_hbm, ps_hbm, dst_hbm, src_hbm, gd_hbm, gs_hbm, di, si, bd, bs, sem1, sem2):
        wid = lax.axis_index("s") * NC + lax.axis_index("c")
        base = wid * EPT

        def chunk(ci, carry):
            off = base + ci * CH
            pltpu.sync_copy(dst_hbm.at[pl.ds(off, CH)], di)
            pltpu.sync_copy(src_hbm.at[pl.ds(off, CH)], si)
            cp1 = pltpu.async_copy(pd_hbm.at[di], bd, sem1)
            cp2 = pltpu.async_copy(ps_hbm.at[si], bs, sem2)
            cp1.wait()
            cp2.wait()
            pltpu.sync_copy(bd, gd_hbm.at[pl.ds(off, CH)])
            pltpu.sync_copy(bs, gs_hbm.at[pl.ds(off, CH)])
            return carry

        lax.fori_loop(0, NCHK, chunk, 0)

    return k(pd, ps, dstp, srcp)


def _tc_logits(gd, gs, eap, a1e, a2):
    def body(gd_ref, gs_ref, ea_ref, a1e_ref, a2_ref, e_ref, m_ref):
        i = pl.program_id(0)
        t = gd_ref[...] + gs_ref[...] + jnp.dot(
            ea_ref[...], a1e_ref[...], preferred_element_type=jnp.float32)
        t = jnp.tanh(t)
        e = jnp.sum(t * a2_ref[...], axis=1, keepdims=True)
        e = jnp.where(e > 0, e, 0.2 * e)
        rows = i * BEDGE + lax.broadcasted_iota(jnp.int32, (BEDGE, 1), 0)
        e = jnp.where(rows < E, e, -1e30)
        e_ref[...] = e
        bm = jnp.broadcast_to(jnp.max(e), (1, D))
        m_ref[...] = jnp.where(i == 0, bm, jnp.maximum(m_ref[...], bm))

    return pl.pallas_call(
        body,
        grid=(GE,),
        in_specs=[pl.BlockSpec((BEDGE, D), lambda i: (i, 0)),
                  pl.BlockSpec((BEDGE, D), lambda i: (i, 0)),
                  pl.BlockSpec((BEDGE, EDIM), lambda i: (i, 0)),
                  pl.BlockSpec((EDIM, D), lambda i: (0, 0)),
                  pl.BlockSpec((1, D), lambda i: (0, 0))],
        out_specs=[pl.BlockSpec((BEDGE, 1), lambda i: (i, 0)),
                   pl.BlockSpec((1, D), lambda i: (0, 0))],
        out_shape=[jax.ShapeDtypeStruct((EP, 1), jnp.float32),
                   jax.ShapeDtypeStruct((1, D), jnp.float32)],
    )(gd, gs, eap, a1e, a2)


def _sc_softmax_denom(e, m, dstp):
    @functools.partial(
        pl.kernel,
        mesh=_mesh(),
        compiler_params=pltpu.CompilerParams(needs_layout_passes=False),
        out_type=[jax.ShapeDtypeStruct((EP,), jnp.float32),
                  jax.ShapeDtypeStruct((NC, NPAD), jnp.float32)],
        scratch_types=[
            pltpu.VMEM((CH,), jnp.float32),
            pltpu.VMEM((CH,), jnp.int32),
            pltpu.VMEM((CH,), jnp.float32),
            pltpu.VMEM((D,), jnp.float32),
            pltpu.VMEM((1024,), jnp.float32),
            pltpu.VMEM_SHARED((NPAD,), jnp.float32),
        ],
    )
    def k(e_hbm, m_hbm, dst_hbm, ex_hbm, sp_hbm, ev, di, exv, mv, zv, s_sh):
        c = lax.axis_index("c")
        s = lax.axis_index("s")
        wid = s * NC + c

        @pl.when(s == 0)
        def _init():
            def zb(i, carry):
                zv[pl.ds(i * 16, 16)] = jnp.zeros((16,), jnp.float32)
                return carry
            lax.fori_loop(0, 1024 // 16, zb, 0)

            def zcopy(i, carry):
                pltpu.sync_copy(zv, s_sh.at[pl.ds(i * 1024, 1024)])
                return carry
            lax.fori_loop(0, NPAD // 1024, zcopy, 0)

        plsc.subcore_barrier()
        pltpu.sync_copy(m_hbm.at[0], mv)
        base = wid * EPT

        def chunk(ci, carry):
            off = base + ci * CH
            pltpu.sync_copy(e_hbm.at[pl.ds(off, CH)], ev)
            pltpu.sync_copy(dst_hbm.at[pl.ds(off, CH)], di)
            mb = mv[pl.ds(0, 16)]
            for j in range(CH // 16):
                sl = pl.ds(j * 16, 16)
                exv[sl] = jnp.exp(ev[sl] - mb)
            pltpu.sync_copy(exv, ex_hbm.at[pl.ds(off, CH)])
            pltpu.sync_copy(exv, s_sh.at[di], add=True)
            return carry

        lax.fori_loop(0, NCHK, chunk, 0)
        plsc.subcore_barrier()

        @pl.when(s == 0)
        def _flush():
            pltpu.sync_copy(s_sh, sp_hbm.at[c])

    return k(e, m, dstp)


def _sc_aggregate(ex, sp, dstp, srcp, pn):
    @functools.partial(
        pl.kernel,
        mesh=_mesh(),
        compiler_params=pltpu.CompilerParams(needs_layout_passes=False),
        out_type=jax.ShapeDtypeStruct((NC, N, D), jnp.float32),
        scratch_types=[
            pltpu.VMEM((NPAD,), jnp.float32),
            pltpu.VMEM((NPAD,), jnp.float32),
            pltpu.VMEM((CH,), jnp.float32),
            pltpu.VMEM((CH,), jnp.int32),
            pltpu.VMEM((CH,), jnp.int32),
            pltpu.VMEM((CH,), jnp.float32),
            pltpu.VMEM((CH, D), jnp.float32),
            pltpu.VMEM_SHARED((NPAD, D), jnp.float32),
            pltpu.SemaphoreType.DMA,
        ],
    )
    def k(ex_hbm, sp_hbm, dst_hbm, src_hbm, pn_hbm, agg_hbm,
          st, tmp, exv, di, si, alv, rowb, agg_sh, sem):
        c = lax.axis_index("c")
        s = lax.axis_index("s")
        wid = s * NC + c

        @pl.when(s == 0)
        def _init():
            def zrow(kk, carry):
                for j in range(D // 16):
                    rowb[kk, pl.ds(j * 16, 16)] = jnp.zeros((16,), jnp.float32)
                return carry
            lax.fori_loop(0, CH, zrow, 0)

            def zcopy(i, carry):
                pltpu.sync_copy(rowb, agg_sh.at[pl.ds(i * CH, CH)])
                return carry
            lax.fori_loop(0, NPAD // CH, zcopy, 0)

        plsc.subcore_barrier()
        pltpu.sync_copy(sp_hbm.at[0], st)
        pltpu.sync_copy(sp_hbm.at[1], tmp)

        def addv(i, carry):
            sl = pl.ds(i * 16, 16)
            st[sl] = st[sl] + tmp[sl]
            return carry
        lax.fori_loop(0, NPAD // 16, addv, 0)

        base = wid * EPT

        def chunk(ci, carry):
            off = base + ci * CH
            pltpu.sync_copy(ex_hbm.at[pl.ds(off, CH)], exv)
            pltpu.sync_copy(dst_hbm.at[pl.ds(off, CH)], di)
            pltpu.sync_copy(src_hbm.at[pl.ds(off, CH)], si)
            pltpu.async_copy(pn_hbm.at[si], rowb, sem).wait()
            for j in range(CH // 16):
                sl = pl.ds(j * 16, 16)
                sg = plsc.load_gather(st, [di[sl]])
                alv[sl] = exv[sl] / (sg + 1e-16)

            def scale(kk, carry2):
                av = plsc.load_gather(alv, [jnp.broadcast_to(kk, (16,))])
                for j in range(D // 16):
                    sl = pl.ds(j * 16, 16)
                    rowb[kk, sl] = rowb[kk, sl] * av
                return carry2
            lax.fori_loop(0, CH, scale, 0)
            pltpu.sync_copy(rowb, agg_sh.at[di], add=True)
            return carry

        lax.fori_loop(0, NCHK, chunk, 0)
        plsc.subcore_barrier()

        @pl.when(s == 0)
        def _flush():
            pltpu.sync_copy(agg_sh.at[pl.ds(0, N)], agg_hbm.at[c])

    return k(ex, sp, dstp, srcp, pn)


def _tc_combine_a(sx, aggp, b):
    def body(sx_ref, ag_ref, b_ref, hp_ref, ssum_ref, ssq_ref):
        i = pl.program_id(0)
        v = sx_ref[...] + ag_ref[0] + ag_ref[1] + b_ref[...]
        hp_ref[...] = v
        bs = jnp.broadcast_to(jnp.sum(v, axis=0, keepdims=True), (8, D))
        bq = jnp.broadcast_to(jnp.sum(v * v, axis=0, keepdims=True), (8, D))
        ssum_ref[...] = jnp.where(i == 0, bs, ssum_ref[...] + bs)
        ssq_ref[...] = jnp.where(i == 0, bq, ssq_ref[...] + bq)

    return pl.pallas_call(
        body,
        grid=(GN,),
        in_specs=[pl.BlockSpec((BNODE, D), lambda i: (i, 0)),
                  pl.BlockSpec((NC, BNODE, D), lambda i: (0, i, 0)),
                  pl.BlockSpec((1, D), lambda i: (0, 0))],
        out_specs=[pl.BlockSpec((BNODE, D), lambda i: (i, 0)),
                   pl.BlockSpec((8, D), lambda i: (0, 0)),
                   pl.BlockSpec((8, D), lambda i: (0, 0))],
        out_shape=[jax.ShapeDtypeStruct((N, D), jnp.float32),
                   jax.ShapeDtypeStruct((8, D), jnp.float32),
                   jax.ShapeDtypeStruct((8, D), jnp.float32)],
    )(sx, aggp, b)


def _tc_combine_b(hp, ssum, ssq, g, be, relu):
    def body(hp_ref, ssum_ref, ssq_ref, g_ref, be_ref, out_ref):
        mu = ssum_ref[0:1, :] * (1.0 / N)
        var = ssq_ref[0:1, :] * (1.0 / N) - mu * mu
        inv = lax.rsqrt(var + 1e-5)
        y = (hp_ref[...] - mu) * inv * g_ref[...] + be_ref[...]
        if relu:
            y = jnp.maximum(y, 0.0)
        out_ref[...] = y

    return pl.pallas_call(
        body,
        grid=(GN,),
        in_specs=[pl.BlockSpec((BNODE, D), lambda i: (i, 0)),
                  pl.BlockSpec((8, D), lambda i: (0, 0)),
                  pl.BlockSpec((8, D), lambda i: (0, 0)),
                  pl.BlockSpec((1, D), lambda i: (0, 0)),
                  pl.BlockSpec((1, D), lambda i: (0, 0))],
        out_specs=pl.BlockSpec((BNODE, D), lambda i: (i, 0)),
        out_shape=jax.ShapeDtypeStruct((N, D), jnp.float32),
    )(hp, ssum, ssq, g, be)


def _tc_head(h, wc, bc):
    def body(h_ref, wc_ref, bc_ref, out_ref):
        lg = jnp.dot(h_ref[...], wc_ref[...],
                     preferred_element_type=jnp.float32) + bc_ref[...]
        z = lg - jnp.max(lg, axis=1, keepdims=True)
        lse = jnp.log(jnp.sum(jnp.exp(z), axis=1, keepdims=True))
        out_ref[...] = z - lse

    return pl.pallas_call(
        body,
        grid=(GN,),
        in_specs=[pl.BlockSpec((BNODE, D), lambda i: (i, 0)),
                  pl.BlockSpec((D, D), lambda i: (0, 0)),
                  pl.BlockSpec((1, D), lambda i: (0, 0))],
        out_specs=pl.BlockSpec((BNODE, D), lambda i: (i, 0)),
        out_shape=jax.ShapeDtypeStruct((N, D), jnp.float32),
    )(h, wc, bc)


def kernel(x, edge_index, edge_attr, Wself, Wnbr, Att1, Att2, bias, gamma, beta, Wc, bc):
    src = edge_index[0]
    dst = edge_index[1]
    pad = EP - E
    srcp = jnp.concatenate([src, jnp.zeros((pad,), jnp.int32)])
    dstp = jnp.concatenate([dst, jnp.zeros((pad,), jnp.int32)])
    eap = jnp.concatenate(
        [edge_attr, jnp.zeros((pad, EDIM), jnp.float32)], axis=0)

    h = x
    for l in range(L):
        a1 = Att1[l]
        wcat = jnp.concatenate(
            [a1[0:D], a1[D:2 * D], Wnbr[l], Wself[l]], axis=1)
        a1e = a1[2 * D:2 * D + EDIM]
        a2 = Att2[l].reshape(1, D)
        pd, ps, pn, sx = _tc_proj(h, wcat)
        gd, gs = _sc_gather(pd, ps, dstp, srcp)
        e2, m = _tc_logits(gd, gs, eap, a1e, a2)
        ex, sp = _sc_softmax_denom(e2.reshape(EP), m, dstp)
        aggp = _sc_aggregate(ex, sp, dstp, srcp, pn)
        hp, ssum, ssq = _tc_combine_a(sx, aggp, bias[l].reshape(1, D))
        h = _tc_combine_b(hp, ssum, ssq, gamma[l].reshape(1, D),
                          beta[l].reshape(1, D), l < L - 1)
    return _tc_head(h, Wc, bc.reshape(1, D))
